# baseline (speedup 1.0000x reference)
import jax
import jax.numpy as jnp
from jax import lax
from jax.experimental import pallas as pl
from jax.experimental.pallas import tpu as pltpu
from jax.experimental.pallas import tpu_sc as plsc

N_NODES = 10000
FEAT = 128
HF = FEAT // 2
N_EDGES = 320000

NC = 2
NS = 16
E_PER_T = N_EDGES // NS
CHUNK = 80
N_CHUNKS = E_PER_T // CHUNK
N_PAD = 10240
ROWS_PER_TILE = N_PAD // NS
ZCH = 128
CW = 16


def _sc_agg(want_count: bool, NBUF: int = 5):
  LAG = NBUF // 2
  mesh = plsc.VectorSubcoreMesh(core_axis_name="c", subcore_axis_name="s")

  out_type = [jax.ShapeDtypeStruct((NC, N_PAD, FEAT), jnp.float32)]
  scratch = [
      pltpu.VMEM((E_PER_T,), jnp.int32),
      pltpu.VMEM((E_PER_T,), jnp.int32),
      pltpu.VMEM((NBUF, CHUNK, HF), jnp.float32),
      pltpu.VMEM((ZCH, HF), jnp.float32),
      pltpu.VMEM_SHARED((N_PAD, HF), jnp.float32),
  ]
  scratch += [pltpu.SemaphoreType.DMA] * (2 * NBUF)
  if want_count:
    out_type.append(jax.ShapeDtypeStruct((N_PAD, FEAT), jnp.float32))
    scratch += [
        pltpu.VMEM((CHUNK, CW), jnp.float32),
        pltpu.VMEM_SHARED((N_PAD, CW), jnp.float32),
        pltpu.SemaphoreType.DMA,
    ]

  def body(table_hbm, src_hbm, dst_hbm, zeros_hbm, zcol_hbm, ones_hbm,
           *rest):
    if want_count:
      part_hbm, cnt_hbm = rest[0], rest[1]
      rest = rest[2:]
    else:
      part_hbm = rest[0]
      rest = rest[1:]
    idx_s, idx_d, rows_v, zbuf_v, acc_sh = rest[:5]
    semg = rest[5:5 + NBUF]
    sems = rest[5 + NBUF:5 + 2 * NBUF]
    if want_count:
      ones_v, cnt_sh, semc = rest[5 + 2 * NBUF:]

    c = lax.axis_index("c")
    s = lax.axis_index("s")
    row0 = s * ROWS_PER_TILE

    def start_gather(chunk, b):
      pltpu.async_copy(table_hbm.at[idx_s.at[pl.ds(chunk * CHUNK, CHUNK)]],
                       rows_v.at[b], semg[b])

    def wait_gather(b):
      pltpu.make_async_copy(table_hbm.at[idx_s.at[pl.ds(0, CHUNK)]],
                            rows_v.at[b], semg[b]).wait()

    def start_scatter(chunk, b):
      pltpu.async_copy(rows_v.at[b],
                       acc_sh.at[idx_d.at[pl.ds(chunk * CHUNK, CHUNK)]],
                       sems[b], add=True)
      if want_count:
        @pl.when(c == 0)
        def _():
          pltpu.async_copy(ones_v,
                           cnt_sh.at[idx_d.at[pl.ds(chunk * CHUNK, CHUNK)]],
                           semc, add=True)

    def wait_scatter(b):
      pltpu.make_async_copy(rows_v.at[b], acc_sh.at[idx_d.at[pl.ds(0, CHUNK)]],
                            sems[b]).wait()
      if want_count:
        @pl.when(c == 0)
        def _():
          pltpu.make_async_copy(ones_v, cnt_sh.at[idx_d.at[pl.ds(0, CHUNK)]],
                                semc).wait()

    pltpu.sync_copy(src_hbm.at[pl.ds(c * N_EDGES + s * E_PER_T, E_PER_T)],
                    idx_s)
    for b in range(LAG):
      start_gather(b, b)
    pltpu.sync_copy(dst_hbm.at[pl.ds(s * E_PER_T, E_PER_T)], idx_d)

    pltpu.sync_copy(zeros_hbm, zbuf_v)
    for z in range(ROWS_PER_TILE // ZCH):
      pltpu.sync_copy(zbuf_v, acc_sh.at[pl.ds(row0 + z * ZCH, ZCH)])
    if want_count:
      pltpu.sync_copy(ones_hbm, ones_v)

      @pl.when(c == 0)
      def _():
        pltpu.sync_copy(zcol_hbm, cnt_sh.at[pl.ds(row0, ROWS_PER_TILE)])
    plsc.subcore_barrier()

    def outer_body(k, carry):
      for b in range(NBUF):
        chunk = k * NBUF + b
        wait_gather(b)
        start_scatter(chunk, b)
        nxt = chunk + LAG
        b2 = (b + LAG) % NBUF

        @pl.when(nxt < N_CHUNKS)
        def _():
          @pl.when(nxt >= NBUF)
          def _():
            wait_scatter(b2)
          start_gather(nxt, b2)
      return carry

    lax.fori_loop(0, N_CHUNKS // NBUF, outer_body, 0)
    for b in range(NBUF):
      wait_scatter(b)
    plsc.subcore_barrier()

    pltpu.sync_copy(acc_sh.at[pl.ds(row0, ROWS_PER_TILE)],
                    part_hbm.at[c, pl.ds(row0, ROWS_PER_TILE), pl.ds(0, HF)])
    if want_count:
      @pl.when(c == 0)
      def _():
        pltpu.sync_copy(cnt_sh.at[pl.ds(row0, ROWS_PER_TILE)],
                        cnt_hbm.at[pl.ds(row0, ROWS_PER_TILE), pl.ds(0, CW)])

  return pl.kernel(body, out_type=tuple(out_type), mesh=mesh,
                   scratch_types=scratch,
                   compiler_params=pltpu.CompilerParams(
                       use_tc_tiling_on_sc=False))


_sc_agg_count = _sc_agg(True, NBUF=5)
_sc_agg_plain = _sc_agg(False, NBUF=5)


BN = 1000


def _tc_root_body(x_ref, wr_ref, b_ref, out_ref):
  out_ref[...] = (jnp.dot(x_ref[...], wr_ref[...],
                          preferred_element_type=jnp.float32) + b_ref[...])


def _tc_root(x, wrT, b):
  grid = (N_NODES // BN,)
  return pl.pallas_call(
      _tc_root_body,
      grid=grid,
      in_specs=[
          pl.BlockSpec((BN, FEAT), lambda i: (i, 0)),
          pl.BlockSpec((FEAT, FEAT), lambda i: (0, 0)),
          pl.BlockSpec((1, FEAT), lambda i: (0, 0)),
      ],
      out_specs=pl.BlockSpec((BN, FEAT), lambda i: (i, 0)),
      out_shape=jax.ShapeDtypeStruct((N_NODES, FEAT), jnp.float32),
  )(x, wrT, b)


def _tc_layer1_body(part_ref, cnt_ref, pre_ref, wl_ref, out_ref, inv_ref):
  inv = 1.0 / jnp.maximum(cnt_ref[:, :1], 1.0)
  agg = jnp.concatenate([part_ref[0, :, :HF], part_ref[1, :, :HF]],
                        axis=1) * inv
  o = (jnp.dot(agg, wl_ref[...], preferred_element_type=jnp.float32)
       + pre_ref[...])
  out_ref[...] = jnp.maximum(o, 0.0)
  inv_ref[...] = inv


def _tc_layer2_body(part_ref, inv_ref, pre_ref, wl_ref, out_ref):
  agg = jnp.concatenate([part_ref[0, :, :HF], part_ref[1, :, :HF]],
                        axis=1) * inv_ref[...]
  out_ref[...] = (jnp.dot(agg, wl_ref[...], preferred_element_type=jnp.float32)
                  + pre_ref[...])


def _tc_layer1(part, cnt, pre, wlT):
  grid = (N_NODES // BN,)
  return pl.pallas_call(
      _tc_layer1_body,
      grid=grid,
      in_specs=[
          pl.BlockSpec((NC, BN, FEAT), lambda i: (0, i, 0)),
          pl.BlockSpec((BN, FEAT), lambda i: (i, 0)),
          pl.BlockSpec((BN, FEAT), lambda i: (i, 0)),
          pl.BlockSpec((FEAT, FEAT), lambda i: (0, 0)),
      ],
      out_specs=[
          pl.BlockSpec((BN, FEAT), lambda i: (i, 0)),
          pl.BlockSpec((BN, 1), lambda i: (i, 0)),
      ],
      out_shape=[
          jax.ShapeDtypeStruct((N_NODES, FEAT), jnp.float32),
          jax.ShapeDtypeStruct((N_NODES, 1), jnp.float32),
      ],
  )(part, cnt, pre, wlT)


def _tc_layer2(part, inv, pre, wlT):
  grid = (N_NODES // BN,)
  return pl.pallas_call(
      _tc_layer2_body,
      grid=grid,
      in_specs=[
          pl.BlockSpec((NC, BN, FEAT), lambda i: (0, i, 0)),
          pl.BlockSpec((BN, 1), lambda i: (i, 0)),
          pl.BlockSpec((BN, FEAT), lambda i: (i, 0)),
          pl.BlockSpec((FEAT, FEAT), lambda i: (0, 0)),
      ],
      out_specs=pl.BlockSpec((BN, FEAT), lambda i: (i, 0)),
      out_shape=jax.ShapeDtypeStruct((N_NODES, FEAT), jnp.float32),
  )(part, inv, pre, wlT)


@jax.jit
def kernel(x, edge_index, W1_l, W1_r, b1, W2_l, W2_r, b2):
  src = edge_index[0].astype(jnp.int32)
  dst = edge_index[1].astype(jnp.int32)
  srcg = jnp.concatenate([src * 2, src * 2 + 1])
  zeros2d = jnp.zeros((ZCH, HF), jnp.float32)
  zcol = jnp.zeros((ROWS_PER_TILE, CW), jnp.float32)
  ones = jnp.ones((CHUNK, CW), jnp.float32)

  xr = x.reshape(2 * N_NODES, HF)
  part1, cnt = _sc_agg_count(xr, srcg, dst, zeros2d, zcol, ones)
  pre1 = _tc_root(x, W1_r.T, b1.reshape(1, FEAT))
  h, inv = _tc_layer1(part1, cnt, pre1, W1_l.T)
  hr = h.reshape(2 * N_NODES, HF)
  (part2,) = _sc_agg_plain(hr, srcg, dst, zeros2d, zcol, ones)
  pre2 = _tc_root(h, W2_r.T, b2.reshape(1, FEAT))
  z = _tc_layer2(part2, inv, pre2, W2_l.T)
  return z


# speedup vs baseline: 1.2110x; 1.2110x over previous
import jax
import jax.numpy as jnp
from jax import lax
from jax.experimental import pallas as pl
from jax.experimental.pallas import tpu as pltpu
from jax.experimental.pallas import tpu_sc as plsc

N_NODES = 10000
FEAT = 128
HF = FEAT // 2
N_EDGES = 320000

NC = 2
NS = 16
E_PER_T = N_EDGES // NS
CHUNK = 80
N_CHUNKS = E_PER_T // CHUNK
N_PAD = 10240
ROWS_PER_TILE = N_PAD // NS
ZCH = 128
CW = 16


def _sc_agg(want_count: bool, NBUF: int = 5):
  LAG = 3
  mesh = plsc.VectorSubcoreMesh(core_axis_name="c", subcore_axis_name="s")

  out_type = [jax.ShapeDtypeStruct((NC, N_PAD, FEAT), jnp.float32)]
  scratch = [
      pltpu.VMEM((E_PER_T,), jnp.int32),
      pltpu.VMEM((E_PER_T,), jnp.int32),
      pltpu.VMEM((NBUF, CHUNK, HF), jnp.float32),
      pltpu.VMEM((ZCH, HF), jnp.float32),
      pltpu.VMEM_SHARED((N_PAD, HF), jnp.float32),
  ]
  scratch += [pltpu.SemaphoreType.DMA] * (2 * NBUF)
  if want_count:
    out_type.append(jax.ShapeDtypeStruct((N_PAD, FEAT), jnp.float32))
    scratch += [
        pltpu.VMEM((CHUNK, CW), jnp.float32),
        pltpu.VMEM_SHARED((N_PAD, CW), jnp.float32),
        pltpu.SemaphoreType.DMA,
    ]

  def body(table_hbm, src_hbm, dst_hbm, zeros_hbm, zcol_hbm, ones_hbm,
           *rest):
    if want_count:
      part_hbm, cnt_hbm = rest[0], rest[1]
      rest = rest[2:]
    else:
      part_hbm = rest[0]
      rest = rest[1:]
    idx_s, idx_d, rows_v, zbuf_v, acc_sh = rest[:5]
    semg = rest[5:5 + NBUF]
    sems = rest[5 + NBUF:5 + 2 * NBUF]
    if want_count:
      ones_v, cnt_sh, semc = rest[5 + 2 * NBUF:]

    c = lax.axis_index("c")
    s = lax.axis_index("s")
    row0 = s * ROWS_PER_TILE

    def start_gather(chunk, b):
      pltpu.async_copy(table_hbm.at[idx_s.at[pl.ds(chunk * CHUNK, CHUNK)]],
                       rows_v.at[b], semg[b])

    def wait_gather(b):
      pltpu.make_async_copy(table_hbm.at[idx_s.at[pl.ds(0, CHUNK)]],
                            rows_v.at[b], semg[b]).wait()

    def start_scatter(chunk, b):
      pltpu.async_copy(rows_v.at[b],
                       acc_sh.at[idx_d.at[pl.ds(chunk * CHUNK, CHUNK)]],
                       sems[b], add=True)
      if want_count:
        @pl.when(c == 0)
        def _():
          pltpu.async_copy(ones_v,
                           cnt_sh.at[idx_d.at[pl.ds(chunk * CHUNK, CHUNK)]],
                           semc, add=True)

    def wait_scatter(b):
      pltpu.make_async_copy(rows_v.at[b], acc_sh.at[idx_d.at[pl.ds(0, CHUNK)]],
                            sems[b]).wait()
      if want_count:
        @pl.when(c == 0)
        def _():
          pltpu.make_async_copy(ones_v, cnt_sh.at[idx_d.at[pl.ds(0, CHUNK)]],
                                semc).wait()

    pltpu.sync_copy(src_hbm.at[pl.ds(c * N_EDGES + s * E_PER_T, E_PER_T)],
                    idx_s)
    for b in range(LAG):
      start_gather(b, b)
    pltpu.sync_copy(dst_hbm.at[pl.ds(s * E_PER_T, E_PER_T)], idx_d)

    pltpu.sync_copy(zeros_hbm, zbuf_v)
    for z in range(ROWS_PER_TILE // ZCH):
      pltpu.sync_copy(zbuf_v, acc_sh.at[pl.ds(row0 + z * ZCH, ZCH)])
    if want_count:
      pltpu.sync_copy(ones_hbm, ones_v)

      @pl.when(c == 0)
      def _():
        pltpu.sync_copy(zcol_hbm, cnt_sh.at[pl.ds(row0, ROWS_PER_TILE)])
    plsc.subcore_barrier()

    def outer_body(k, carry):
      for b in range(NBUF):
        chunk = k * NBUF + b
        wait_gather(b)
        start_scatter(chunk, b)
        nxt = chunk + LAG
        b2 = (b + LAG) % NBUF

        @pl.when(nxt < N_CHUNKS)
        def _():
          @pl.when(nxt >= NBUF)
          def _():
            wait_scatter(b2)
          start_gather(nxt, b2)
      return carry

    lax.fori_loop(0, N_CHUNKS // NBUF, outer_body, 0)
    for b in range(NBUF):
      wait_scatter(b)
    plsc.subcore_barrier()

    pltpu.sync_copy(acc_sh.at[pl.ds(row0, ROWS_PER_TILE)],
                    part_hbm.at[c, pl.ds(row0, ROWS_PER_TILE), pl.ds(0, HF)])
    if want_count:
      @pl.when(c == 0)
      def _():
        pltpu.sync_copy(cnt_sh.at[pl.ds(row0, ROWS_PER_TILE)],
                        cnt_hbm.at[pl.ds(row0, ROWS_PER_TILE), pl.ds(0, CW)])

  return pl.kernel(body, out_type=tuple(out_type), mesh=mesh,
                   scratch_types=scratch,
                   compiler_params=pltpu.CompilerParams(
                       use_tc_tiling_on_sc=False))


_sc_agg_count = _sc_agg(True, NBUF=5)
_sc_agg_plain = _sc_agg(False, NBUF=5)


BN = 1000


def _tc_root_body(x_ref, wr_ref, b_ref, out_ref):
  out_ref[...] = (jnp.dot(x_ref[...], wr_ref[...],
                          preferred_element_type=jnp.float32) + b_ref[...])


def _tc_root(x, wrT, b):
  grid = (N_NODES // BN,)
  return pl.pallas_call(
      _tc_root_body,
      grid=grid,
      in_specs=[
          pl.BlockSpec((BN, FEAT), lambda i: (i, 0)),
          pl.BlockSpec((FEAT, FEAT), lambda i: (0, 0)),
          pl.BlockSpec((1, FEAT), lambda i: (0, 0)),
      ],
      out_specs=pl.BlockSpec((BN, FEAT), lambda i: (i, 0)),
      out_shape=jax.ShapeDtypeStruct((N_NODES, FEAT), jnp.float32),
  )(x, wrT, b)


def _tc_layer1_body(part_ref, cnt_ref, pre_ref, wl_ref, out_ref, inv_ref):
  inv = 1.0 / jnp.maximum(cnt_ref[:, :1], 1.0)
  agg = jnp.concatenate([part_ref[0, :, :HF], part_ref[1, :, :HF]],
                        axis=1) * inv
  o = (jnp.dot(agg, wl_ref[...], preferred_element_type=jnp.float32)
       + pre_ref[...])
  out_ref[...] = jnp.maximum(o, 0.0)
  inv_ref[...] = inv


def _tc_layer2_body(part_ref, inv_ref, pre_ref, wl_ref, out_ref):
  agg = jnp.concatenate([part_ref[0, :, :HF], part_ref[1, :, :HF]],
                        axis=1) * inv_ref[...]
  out_ref[...] = (jnp.dot(agg, wl_ref[...], preferred_element_type=jnp.float32)
                  + pre_ref[...])


def _tc_layer1(part, cnt, pre, wlT):
  grid = (N_NODES // BN,)
  return pl.pallas_call(
      _tc_layer1_body,
      grid=grid,
      in_specs=[
          pl.BlockSpec((NC, BN, FEAT), lambda i: (0, i, 0)),
          pl.BlockSpec((BN, FEAT), lambda i: (i, 0)),
          pl.BlockSpec((BN, FEAT), lambda i: (i, 0)),
          pl.BlockSpec((FEAT, FEAT), lambda i: (0, 0)),
      ],
      out_specs=[
          pl.BlockSpec((BN, FEAT), lambda i: (i, 0)),
          pl.BlockSpec((BN, 1), lambda i: (i, 0)),
      ],
      out_shape=[
          jax.ShapeDtypeStruct((N_NODES, FEAT), jnp.float32),
          jax.ShapeDtypeStruct((N_NODES, 1), jnp.float32),
      ],
  )(part, cnt, pre, wlT)


def _tc_layer2(part, inv, pre, wlT):
  grid = (N_NODES // BN,)
  return pl.pallas_call(
      _tc_layer2_body,
      grid=grid,
      in_specs=[
          pl.BlockSpec((NC, BN, FEAT), lambda i: (0, i, 0)),
          pl.BlockSpec((BN, 1), lambda i: (i, 0)),
          pl.BlockSpec((BN, FEAT), lambda i: (i, 0)),
          pl.BlockSpec((FEAT, FEAT), lambda i: (0, 0)),
      ],
      out_specs=pl.BlockSpec((BN, FEAT), lambda i: (i, 0)),
      out_shape=jax.ShapeDtypeStruct((N_NODES, FEAT), jnp.float32),
  )(part, inv, pre, wlT)


@jax.jit
def kernel(x, edge_index, W1_l, W1_r, b1, W2_l, W2_r, b2):
  src = edge_index[0].astype(jnp.int32)
  dst = edge_index[1].astype(jnp.int32)
  srcg = jnp.concatenate([src * 2, src * 2 + 1])
  zeros2d = jnp.zeros((ZCH, HF), jnp.float32)
  zcol = jnp.zeros((ROWS_PER_TILE, CW), jnp.float32)
  ones = jnp.ones((CHUNK, CW), jnp.float32)

  xr = x.reshape(2 * N_NODES, HF)
  part1, cnt = _sc_agg_count(xr, srcg, dst, zeros2d, zcol, ones)
  pre1 = _tc_root(x, W1_r.T, b1.reshape(1, FEAT))
  h, inv = _tc_layer1(part1, cnt, pre1, W1_l.T)
  hr = h.reshape(2 * N_NODES, HF)
  (part2,) = _sc_agg_plain(hr, srcg, dst, zeros2d, zcol, ones)
  pre2 = _tc_root(h, W2_r.T, b2.reshape(1, FEAT))
  z = _tc_layer2(part2, inv, pre2, W2_l.T)
  return z


# speedup vs baseline: 1.2751x; 1.0530x over previous
import jax
import jax.numpy as jnp
from jax import lax
from jax.experimental import pallas as pl
from jax.experimental.pallas import tpu as pltpu
from jax.experimental.pallas import tpu_sc as plsc

N_NODES = 10000
FEAT = 128
HF = FEAT // 2
N_EDGES = 320000

NC = 2
NS = 16
E_PER_T = N_EDGES // NS
CHUNK = 80
N_CHUNKS = E_PER_T // CHUNK
N_PAD = 10240
ROWS_PER_TILE = N_PAD // NS
ZCH = 128
CW = 16


def _sc_agg(want_count: bool, NBUF: int = 5):
  LAG = 4
  mesh = plsc.VectorSubcoreMesh(core_axis_name="c", subcore_axis_name="s")

  out_type = [jax.ShapeDtypeStruct((NC, N_PAD, FEAT), jnp.float32)]
  scratch = [
      pltpu.VMEM((E_PER_T,), jnp.int32),
      pltpu.VMEM((E_PER_T,), jnp.int32),
      pltpu.VMEM((NBUF, CHUNK, HF), jnp.float32),
      pltpu.VMEM((ZCH, HF), jnp.float32),
      pltpu.VMEM_SHARED((N_PAD, HF), jnp.float32),
  ]
  scratch += [pltpu.SemaphoreType.DMA] * (2 * NBUF)
  if want_count:
    out_type.append(jax.ShapeDtypeStruct((N_PAD, FEAT), jnp.float32))
    scratch += [
        pltpu.VMEM((CHUNK, CW), jnp.float32),
        pltpu.VMEM_SHARED((N_PAD, CW), jnp.float32),
        pltpu.SemaphoreType.DMA,
    ]

  def body(table_hbm, src_hbm, dst_hbm, zeros_hbm, zcol_hbm, ones_hbm,
           *rest):
    if want_count:
      part_hbm, cnt_hbm = rest[0], rest[1]
      rest = rest[2:]
    else:
      part_hbm = rest[0]
      rest = rest[1:]
    idx_s, idx_d, rows_v, zbuf_v, acc_sh = rest[:5]
    semg = rest[5:5 + NBUF]
    sems = rest[5 + NBUF:5 + 2 * NBUF]
    if want_count:
      ones_v, cnt_sh, semc = rest[5 + 2 * NBUF:]

    c = lax.axis_index("c")
    s = lax.axis_index("s")
    row0 = s * ROWS_PER_TILE

    def start_gather(chunk, b):
      pltpu.async_copy(table_hbm.at[idx_s.at[pl.ds(chunk * CHUNK, CHUNK)]],
                       rows_v.at[b], semg[b])

    def wait_gather(b):
      pltpu.make_async_copy(table_hbm.at[idx_s.at[pl.ds(0, CHUNK)]],
                            rows_v.at[b], semg[b]).wait()

    def start_scatter(chunk, b):
      pltpu.async_copy(rows_v.at[b],
                       acc_sh.at[idx_d.at[pl.ds(chunk * CHUNK, CHUNK)]],
                       sems[b], add=True)
      if want_count:
        @pl.when(c == 0)
        def _():
          pltpu.async_copy(ones_v,
                           cnt_sh.at[idx_d.at[pl.ds(chunk * CHUNK, CHUNK)]],
                           semc, add=True)

    def wait_scatter(b):
      pltpu.make_async_copy(rows_v.at[b], acc_sh.at[idx_d.at[pl.ds(0, CHUNK)]],
                            sems[b]).wait()
      if want_count:
        @pl.when(c == 0)
        def _():
          pltpu.make_async_copy(ones_v, cnt_sh.at[idx_d.at[pl.ds(0, CHUNK)]],
                                semc).wait()

    pltpu.sync_copy(src_hbm.at[pl.ds(c * N_EDGES + s * E_PER_T, E_PER_T)],
                    idx_s)
    for b in range(LAG):
      start_gather(b, b)
    pltpu.sync_copy(dst_hbm.at[pl.ds(s * E_PER_T, E_PER_T)], idx_d)

    pltpu.sync_copy(zeros_hbm, zbuf_v)
    for z in range(ROWS_PER_TILE // ZCH):
      pltpu.sync_copy(zbuf_v, acc_sh.at[pl.ds(row0 + z * ZCH, ZCH)])
    if want_count:
      pltpu.sync_copy(ones_hbm, ones_v)

      @pl.when(c == 0)
      def _():
        pltpu.sync_copy(zcol_hbm, cnt_sh.at[pl.ds(row0, ROWS_PER_TILE)])
    plsc.subcore_barrier()

    def outer_body(k, carry):
      for b in range(NBUF):
        chunk = k * NBUF + b
        wait_gather(b)
        start_scatter(chunk, b)
        nxt = chunk + LAG
        b2 = (b + LAG) % NBUF

        @pl.when(nxt < N_CHUNKS)
        def _():
          @pl.when(nxt >= NBUF)
          def _():
            wait_scatter(b2)
          start_gather(nxt, b2)
      return carry

    lax.fori_loop(0, N_CHUNKS // NBUF, outer_body, 0)
    for b in range(NBUF):
      wait_scatter(b)
    plsc.subcore_barrier()

    pltpu.sync_copy(acc_sh.at[pl.ds(row0, ROWS_PER_TILE)],
                    part_hbm.at[c, pl.ds(row0, ROWS_PER_TILE), pl.ds(0, HF)])
    if want_count:
      @pl.when(c == 0)
      def _():
        pltpu.sync_copy(cnt_sh.at[pl.ds(row0, ROWS_PER_TILE)],
                        cnt_hbm.at[pl.ds(row0, ROWS_PER_TILE), pl.ds(0, CW)])

  return pl.kernel(body, out_type=tuple(out_type), mesh=mesh,
                   scratch_types=scratch,
                   compiler_params=pltpu.CompilerParams(
                       use_tc_tiling_on_sc=False))


_sc_agg_count = _sc_agg(True, NBUF=5)
_sc_agg_plain = _sc_agg(False, NBUF=5)


BN = 1000


def _tc_root_body(x_ref, wr_ref, b_ref, out_ref):
  out_ref[...] = (jnp.dot(x_ref[...], wr_ref[...],
                          preferred_element_type=jnp.float32) + b_ref[...])


def _tc_root(x, wrT, b):
  grid = (N_NODES // BN,)
  return pl.pallas_call(
      _tc_root_body,
      grid=grid,
      in_specs=[
          pl.BlockSpec((BN, FEAT), lambda i: (i, 0)),
          pl.BlockSpec((FEAT, FEAT), lambda i: (0, 0)),
          pl.BlockSpec((1, FEAT), lambda i: (0, 0)),
      ],
      out_specs=pl.BlockSpec((BN, FEAT), lambda i: (i, 0)),
      out_shape=jax.ShapeDtypeStruct((N_NODES, FEAT), jnp.float32),
  )(x, wrT, b)


def _tc_layer1_body(part_ref, cnt_ref, pre_ref, wl_ref, out_ref, inv_ref):
  inv = 1.0 / jnp.maximum(cnt_ref[:, :1], 1.0)
  agg = jnp.concatenate([part_ref[0, :, :HF], part_ref[1, :, :HF]],
                        axis=1) * inv
  o = (jnp.dot(agg, wl_ref[...], preferred_element_type=jnp.float32)
       + pre_ref[...])
  out_ref[...] = jnp.maximum(o, 0.0)
  inv_ref[...] = inv


def _tc_layer2_body(part_ref, inv_ref, pre_ref, wl_ref, out_ref):
  agg = jnp.concatenate([part_ref[0, :, :HF], part_ref[1, :, :HF]],
                        axis=1) * inv_ref[...]
  out_ref[...] = (jnp.dot(agg, wl_ref[...], preferred_element_type=jnp.float32)
                  + pre_ref[...])


def _tc_layer1(part, cnt, pre, wlT):
  grid = (N_NODES // BN,)
  return pl.pallas_call(
      _tc_layer1_body,
      grid=grid,
      in_specs=[
          pl.BlockSpec((NC, BN, FEAT), lambda i: (0, i, 0)),
          pl.BlockSpec((BN, FEAT), lambda i: (i, 0)),
          pl.BlockSpec((BN, FEAT), lambda i: (i, 0)),
          pl.BlockSpec((FEAT, FEAT), lambda i: (0, 0)),
      ],
      out_specs=[
          pl.BlockSpec((BN, FEAT), lambda i: (i, 0)),
          pl.BlockSpec((BN, 1), lambda i: (i, 0)),
      ],
      out_shape=[
          jax.ShapeDtypeStruct((N_NODES, FEAT), jnp.float32),
          jax.ShapeDtypeStruct((N_NODES, 1), jnp.float32),
      ],
  )(part, cnt, pre, wlT)


def _tc_layer2(part, inv, pre, wlT):
  grid = (N_NODES // BN,)
  return pl.pallas_call(
      _tc_layer2_body,
      grid=grid,
      in_specs=[
          pl.BlockSpec((NC, BN, FEAT), lambda i: (0, i, 0)),
          pl.BlockSpec((BN, 1), lambda i: (i, 0)),
          pl.BlockSpec((BN, FEAT), lambda i: (i, 0)),
          pl.BlockSpec((FEAT, FEAT), lambda i: (0, 0)),
      ],
      out_specs=pl.BlockSpec((BN, FEAT), lambda i: (i, 0)),
      out_shape=jax.ShapeDtypeStruct((N_NODES, FEAT), jnp.float32),
  )(part, inv, pre, wlT)


@jax.jit
def kernel(x, edge_index, W1_l, W1_r, b1, W2_l, W2_r, b2):
  src = edge_index[0].astype(jnp.int32)
  dst = edge_index[1].astype(jnp.int32)
  srcg = jnp.concatenate([src * 2, src * 2 + 1])
  zeros2d = jnp.zeros((ZCH, HF), jnp.float32)
  zcol = jnp.zeros((ROWS_PER_TILE, CW), jnp.float32)
  ones = jnp.ones((CHUNK, CW), jnp.float32)

  xr = x.reshape(2 * N_NODES, HF)
  part1, cnt = _sc_agg_count(xr, srcg, dst, zeros2d, zcol, ones)
  pre1 = _tc_root(x, W1_r.T, b1.reshape(1, FEAT))
  h, inv = _tc_layer1(part1, cnt, pre1, W1_l.T)
  hr = h.reshape(2 * N_NODES, HF)
  (part2,) = _sc_agg_plain(hr, srcg, dst, zeros2d, zcol, ones)
  pre2 = _tc_root(h, W2_r.T, b2.reshape(1, FEAT))
  z = _tc_layer2(part2, inv, pre2, W2_l.T)
  return z


# speedup vs baseline: 1.2896x; 1.0113x over previous
import jax
import jax.numpy as jnp
from jax import lax
from jax.experimental import pallas as pl
from jax.experimental.pallas import tpu as pltpu
from jax.experimental.pallas import tpu_sc as plsc

N_NODES = 10000
FEAT = 128
HF = FEAT // 2
N_EDGES = 320000

NC = 2
NS = 16
E_PER_T = N_EDGES // NS
CHUNK = 40
N_CHUNKS = E_PER_T // CHUNK
N_PAD = 10240
ROWS_PER_TILE = N_PAD // NS
ZCH = 128
CW = 16


def _sc_agg(want_count: bool, NBUF: int = 5):
  LAG = NBUF - 1
  mesh = plsc.VectorSubcoreMesh(core_axis_name="c", subcore_axis_name="s")

  out_type = [jax.ShapeDtypeStruct((NC, N_PAD, FEAT), jnp.float32)]
  scratch = [
      pltpu.VMEM((E_PER_T,), jnp.int32),
      pltpu.VMEM((E_PER_T,), jnp.int32),
      pltpu.VMEM((NBUF, CHUNK, HF), jnp.float32),
      pltpu.VMEM((ZCH, HF), jnp.float32),
      pltpu.VMEM_SHARED((N_PAD, HF), jnp.float32),
  ]
  scratch += [pltpu.SemaphoreType.DMA] * (2 * NBUF)
  if want_count:
    out_type.append(jax.ShapeDtypeStruct((N_PAD, FEAT), jnp.float32))
    scratch += [
        pltpu.VMEM((CHUNK, CW), jnp.float32),
        pltpu.VMEM_SHARED((N_PAD, CW), jnp.float32),
        pltpu.SemaphoreType.DMA,
    ]

  def body(table_hbm, src_hbm, dst_hbm, zeros_hbm, zcol_hbm, ones_hbm,
           *rest):
    if want_count:
      part_hbm, cnt_hbm = rest[0], rest[1]
      rest = rest[2:]
    else:
      part_hbm = rest[0]
      rest = rest[1:]
    idx_s, idx_d, rows_v, zbuf_v, acc_sh = rest[:5]
    semg = rest[5:5 + NBUF]
    sems = rest[5 + NBUF:5 + 2 * NBUF]
    if want_count:
      ones_v, cnt_sh, semc = rest[5 + 2 * NBUF:]

    c = lax.axis_index("c")
    s = lax.axis_index("s")
    row0 = s * ROWS_PER_TILE

    def start_gather(chunk, b):
      pltpu.async_copy(table_hbm.at[idx_s.at[pl.ds(chunk * CHUNK, CHUNK)]],
                       rows_v.at[b], semg[b])

    def wait_gather(b):
      pltpu.make_async_copy(table_hbm.at[idx_s.at[pl.ds(0, CHUNK)]],
                            rows_v.at[b], semg[b]).wait()

    def start_scatter(chunk, b):
      pltpu.async_copy(rows_v.at[b],
                       acc_sh.at[idx_d.at[pl.ds(chunk * CHUNK, CHUNK)]],
                       sems[b], add=True)
      if want_count:
        @pl.when(c == 0)
        def _():
          pltpu.async_copy(ones_v,
                           cnt_sh.at[idx_d.at[pl.ds(chunk * CHUNK, CHUNK)]],
                           semc, add=True)

    def wait_scatter(b):
      pltpu.make_async_copy(rows_v.at[b], acc_sh.at[idx_d.at[pl.ds(0, CHUNK)]],
                            sems[b]).wait()
      if want_count:
        @pl.when(c == 0)
        def _():
          pltpu.make_async_copy(ones_v, cnt_sh.at[idx_d.at[pl.ds(0, CHUNK)]],
                                semc).wait()

    pltpu.sync_copy(src_hbm.at[pl.ds(c * N_EDGES + s * E_PER_T, E_PER_T)],
                    idx_s)
    for b in range(LAG):
      start_gather(b, b)
    pltpu.sync_copy(dst_hbm.at[pl.ds(s * E_PER_T, E_PER_T)], idx_d)

    pltpu.sync_copy(zeros_hbm, zbuf_v)
    for z in range(ROWS_PER_TILE // ZCH):
      pltpu.sync_copy(zbuf_v, acc_sh.at[pl.ds(row0 + z * ZCH, ZCH)])
    if want_count:
      pltpu.sync_copy(ones_hbm, ones_v)

      @pl.when(c == 0)
      def _():
        pltpu.sync_copy(zcol_hbm, cnt_sh.at[pl.ds(row0, ROWS_PER_TILE)])
    plsc.subcore_barrier()

    def outer_body(k, carry):
      for b in range(NBUF):
        chunk = k * NBUF + b
        wait_gather(b)
        start_scatter(chunk, b)
        nxt = chunk + LAG
        b2 = (b + LAG) % NBUF

        @pl.when(nxt < N_CHUNKS)
        def _():
          @pl.when(nxt >= NBUF)
          def _():
            wait_scatter(b2)
          start_gather(nxt, b2)
      return carry

    lax.fori_loop(0, N_CHUNKS // NBUF, outer_body, 0)
    for b in range(NBUF):
      wait_scatter(b)
    plsc.subcore_barrier()

    pltpu.sync_copy(acc_sh.at[pl.ds(row0, ROWS_PER_TILE)],
                    part_hbm.at[c, pl.ds(row0, ROWS_PER_TILE), pl.ds(0, HF)])
    if want_count:
      @pl.when(c == 0)
      def _():
        pltpu.sync_copy(cnt_sh.at[pl.ds(row0, ROWS_PER_TILE)],
                        cnt_hbm.at[pl.ds(row0, ROWS_PER_TILE), pl.ds(0, CW)])

  return pl.kernel(body, out_type=tuple(out_type), mesh=mesh,
                   scratch_types=scratch,
                   compiler_params=pltpu.CompilerParams(
                       use_tc_tiling_on_sc=False))


_sc_agg_count = _sc_agg(True, NBUF=10)
_sc_agg_plain = _sc_agg(False, NBUF=10)


BN = 1000


def _tc_root_body(x_ref, wr_ref, b_ref, out_ref):
  out_ref[...] = (jnp.dot(x_ref[...], wr_ref[...],
                          preferred_element_type=jnp.float32) + b_ref[...])


def _tc_root(x, wrT, b):
  grid = (N_NODES // BN,)
  return pl.pallas_call(
      _tc_root_body,
      grid=grid,
      in_specs=[
          pl.BlockSpec((BN, FEAT), lambda i: (i, 0)),
          pl.BlockSpec((FEAT, FEAT), lambda i: (0, 0)),
          pl.BlockSpec((1, FEAT), lambda i: (0, 0)),
      ],
      out_specs=pl.BlockSpec((BN, FEAT), lambda i: (i, 0)),
      out_shape=jax.ShapeDtypeStruct((N_NODES, FEAT), jnp.float32),
  )(x, wrT, b)


def _tc_layer1_body(part_ref, cnt_ref, pre_ref, wl_ref, out_ref, inv_ref):
  inv = 1.0 / jnp.maximum(cnt_ref[:, :1], 1.0)
  agg = jnp.concatenate([part_ref[0, :, :HF], part_ref[1, :, :HF]],
                        axis=1) * inv
  o = (jnp.dot(agg, wl_ref[...], preferred_element_type=jnp.float32)
       + pre_ref[...])
  out_ref[...] = jnp.maximum(o, 0.0)
  inv_ref[...] = inv


def _tc_layer2_body(part_ref, inv_ref, pre_ref, wl_ref, out_ref):
  agg = jnp.concatenate([part_ref[0, :, :HF], part_ref[1, :, :HF]],
                        axis=1) * inv_ref[...]
  out_ref[...] = (jnp.dot(agg, wl_ref[...], preferred_element_type=jnp.float32)
                  + pre_ref[...])


def _tc_layer1(part, cnt, pre, wlT):
  grid = (N_NODES // BN,)
  return pl.pallas_call(
      _tc_layer1_body,
      grid=grid,
      in_specs=[
          pl.BlockSpec((NC, BN, FEAT), lambda i: (0, i, 0)),
          pl.BlockSpec((BN, FEAT), lambda i: (i, 0)),
          pl.BlockSpec((BN, FEAT), lambda i: (i, 0)),
          pl.BlockSpec((FEAT, FEAT), lambda i: (0, 0)),
      ],
      out_specs=[
          pl.BlockSpec((BN, FEAT), lambda i: (i, 0)),
          pl.BlockSpec((BN, 1), lambda i: (i, 0)),
      ],
      out_shape=[
          jax.ShapeDtypeStruct((N_NODES, FEAT), jnp.float32),
          jax.ShapeDtypeStruct((N_NODES, 1), jnp.float32),
      ],
  )(part, cnt, pre, wlT)


def _tc_layer2(part, inv, pre, wlT):
  grid = (N_NODES // BN,)
  return pl.pallas_call(
      _tc_layer2_body,
      grid=grid,
      in_specs=[
          pl.BlockSpec((NC, BN, FEAT), lambda i: (0, i, 0)),
          pl.BlockSpec((BN, 1), lambda i: (i, 0)),
          pl.BlockSpec((BN, FEAT), lambda i: (i, 0)),
          pl.BlockSpec((FEAT, FEAT), lambda i: (0, 0)),
      ],
      out_specs=pl.BlockSpec((BN, FEAT), lambda i: (i, 0)),
      out_shape=jax.ShapeDtypeStruct((N_NODES, FEAT), jnp.float32),
  )(part, inv, pre, wlT)


@jax.jit
def kernel(x, edge_index, W1_l, W1_r, b1, W2_l, W2_r, b2):
  src = edge_index[0].astype(jnp.int32)
  dst = edge_index[1].astype(jnp.int32)
  srcg = jnp.concatenate([src * 2, src * 2 + 1])
  zeros2d = jnp.zeros((ZCH, HF), jnp.float32)
  zcol = jnp.zeros((ROWS_PER_TILE, CW), jnp.float32)
  ones = jnp.ones((CHUNK, CW), jnp.float32)

  xr = x.reshape(2 * N_NODES, HF)
  part1, cnt = _sc_agg_count(xr, srcg, dst, zeros2d, zcol, ones)
  pre1 = _tc_root(x, W1_r.T, b1.reshape(1, FEAT))
  h, inv = _tc_layer1(part1, cnt, pre1, W1_l.T)
  hr = h.reshape(2 * N_NODES, HF)
  (part2,) = _sc_agg_plain(hr, srcg, dst, zeros2d, zcol, ones)
  pre2 = _tc_root(h, W2_r.T, b2.reshape(1, FEAT))
  z = _tc_layer2(part2, inv, pre2, W2_l.T)
  return z
